# baseline (speedup 1.0000x reference)
import jax
import jax.numpy as jnp
from jax import lax
from jax.experimental import pallas as pl
from jax.experimental.pallas import tpu as pltpu

_N = 262144
_C = 64
_NP = _N // 2
_NUM_OPS = 16
_NUM_WRITE_OPS = 8

_B = 8192
_G = _NP // _B


def _fused_body(wp_ref, ope_ref, opo_ref, pre_ref, pro_ref, t_ref, o_ref):
    ope = ope_ref[0, 0, :]
    opo = opo_ref[0, 0, :]
    pre = pre_ref[0, 0, :].astype(jnp.float32)
    pro = pro_ref[0, 0, :].astype(jnp.float32)
    pos_e = jnp.full((_B,), -1, dtype=jnp.int32)
    pos_o = jnp.full((_B,), -1, dtype=jnp.int32)
    for k in range(_NUM_OPS):
        tgt_e = jnp.where(k < _NUM_WRITE_OPS, wp_ref[k], -1)
        tgt_o = jnp.where(k < _NUM_WRITE_OPS, wp_ref[k] + _C, -1)
        pos_e = jnp.where(ope == k, tgt_e, pos_e)
        pos_o = jnp.where(opo == k, tgt_o, pos_o)
    lane = lax.broadcasted_iota(jnp.int32, (_B, 128), 1)
    hit = (lane == pos_e[:, None]) | (lane == pos_o[:, None])
    val = jnp.where(lane < _C, pre[:, None], pro[:, None])
    o_ref[...] = jnp.where(hit, val, t_ref[...])


def kernel(tensor, operation, prediction, write_positions):
    t2 = tensor.reshape(_NP, 128)
    op2 = operation.reshape(_NP, 2)
    pr2 = prediction.reshape(_NP, 2)
    ope = op2[:, 0].reshape(_G, 1, _B)
    opo = op2[:, 1].reshape(_G, 1, _B)
    pre = pr2[:, 0].reshape(_G, 1, _B)
    pro = pr2[:, 1].reshape(_G, 1, _B)
    out2 = pl.pallas_call(
        _fused_body,
        grid=(_G,),
        in_specs=[
            pl.BlockSpec(memory_space=pltpu.SMEM),
            pl.BlockSpec((1, 1, _B), lambda i: (i, 0, 0)),
            pl.BlockSpec((1, 1, _B), lambda i: (i, 0, 0)),
            pl.BlockSpec((1, 1, _B), lambda i: (i, 0, 0)),
            pl.BlockSpec((1, 1, _B), lambda i: (i, 0, 0)),
            pl.BlockSpec((_B, 128), lambda i: (i, 0)),
        ],
        out_specs=pl.BlockSpec((_B, 128), lambda i: (i, 0)),
        out_shape=jax.ShapeDtypeStruct((_NP, 128), jnp.float32),
        compiler_params=pltpu.CompilerParams(
            dimension_semantics=("arbitrary",)),
    )(write_positions, ope, opo, pre, pro, t2)
    return out2.reshape(_N, _C)


# speedup vs baseline: 1.6568x; 1.6568x over previous
import functools

import jax
import jax.numpy as jnp
from jax import lax
from jax.experimental import pallas as pl
from jax.experimental.pallas import tpu as pltpu
from jax.experimental.pallas import tpu_sc as plsc

_N = 262144
_C = 64
_NUM_WRITE_OPS = 8

_NC = 2
_NS = 16
_NW = _NC * _NS
_L = 16

_ROWS_PER_W = _N // _NW
_R = 512
_CW = _R * _C
_CHUNKS = _ROWS_PER_W // _R
_PAIRS = _CHUNKS // 2


@functools.partial(
    pl.kernel,
    out_type=jax.ShapeDtypeStruct((_N * _C,), jnp.float32),
    mesh=plsc.VectorSubcoreMesh(core_axis_name="c", subcore_axis_name="s"),
    compiler_params=pltpu.CompilerParams(needs_layout_passes=False),
    scratch_types=(
        [
            pltpu.VMEM((_CW,), jnp.float32),
            pltpu.VMEM((_CW,), jnp.float32),
            pltpu.VMEM((_ROWS_PER_W,), jnp.int32),
            pltpu.VMEM((_ROWS_PER_W,), jnp.int32),
            pltpu.VMEM((_L,), jnp.int32),
        ]
        + [pltpu.SemaphoreType.DMA for _ in range(4)]
    ),
)
def _sc_dispatch(tensor_hbm, op_hbm, pred_hbm, wp_hbm, out_hbm,
                 b0, b1, opbuf, prbuf, wpv, is0, is1, os0, os1):
    bufs = (b0, b1)
    isems = (is0, is1)
    osems = (os0, os1)

    wid = lax.axis_index("s") * _NC + lax.axis_index("c")
    base = wid * _ROWS_PER_W

    pltpu.sync_copy(wp_hbm, wpv)
    pltpu.sync_copy(op_hbm.at[pl.ds(base, _ROWS_PER_W)], opbuf)
    pltpu.sync_copy(pred_hbm.at[pl.ds(base, _ROWS_PER_W)], prbuf)
    wp_vec = wpv[...]

    def start_in(g, slot):
        return pltpu.async_copy(
            tensor_hbm.at[pl.ds((base + g * _R) * _C, _CW)],
            bufs[slot], isems[slot])

    def start_out(g, slot):
        return pltpu.async_copy(
            bufs[slot],
            out_hbm.at[pl.ds((base + g * _R) * _C, _CW)], osems[slot])

    def compute(g, slot):
        buf = bufs[slot]
        lbase = g * _R

        def vec_body(j, c2):
            opv = opbuf[pl.ds(lbase + j * _L, _L)]
            prv = prbuf[pl.ds(lbase + j * _L, _L)].astype(jnp.float32)
            pos = lax.gather(
                wp_vec, opv[:, None],
                lax.GatherDimensionNumbers(
                    offset_dims=(), collapsed_slice_dims=(0,),
                    start_index_map=(0,)),
                slice_sizes=(1,),
                mode=lax.GatherScatterMode.PROMISE_IN_BOUNDS)
            rows = lax.iota(jnp.int32, _L) + j * _L
            idx = rows * _C + pos
            msk = opv < _NUM_WRITE_OPS
            plsc.store_scatter(buf, [idx], prv, mask=msk)
            return c2

        lax.fori_loop(0, _R // _L, vec_body, 0)

    def wait_in(slot):
        pltpu.make_async_copy(
            tensor_hbm.at[pl.ds(base * _C, _CW)], bufs[slot], isems[slot],
        ).wait()

    def wait_out(slot):
        pltpu.make_async_copy(
            bufs[slot], out_hbm.at[pl.ds(base * _C, _CW)], osems[slot],
        ).wait()

    def stage(g, slot):
        wait_in(slot)
        compute(g, slot)
        start_out(g, slot)
        @pl.when(g + 2 < _CHUNKS)
        def _():
            wait_out(slot)
            start_in(g + 2, slot)

    start_in(0, 0)
    start_in(1, 1)

    def pair_body(i, carry):
        g = i * 2
        stage(g, 0)
        stage(g + 1, 1)
        return carry

    lax.fori_loop(0, _PAIRS, pair_body, 0)

    wait_out(0)
    wait_out(1)


def kernel(tensor, operation, prediction, write_positions):
    flat = _sc_dispatch(tensor.reshape(-1), operation, prediction,
                        write_positions)
    return flat.reshape(_N, _C)


# speedup vs baseline: 2.1297x; 1.2854x over previous
import functools

import jax
import jax.numpy as jnp
from jax import lax
from jax.experimental import pallas as pl
from jax.experimental.pallas import tpu as pltpu
from jax.experimental.pallas import tpu_sc as plsc

_N = 262144
_C = 64
_NUM_WRITE_OPS = 8

_NC = 2
_NS = 16
_NW = _NC * _NS
_L = 16

_ROWS_PER_W = _N // _NW
_R = 256
_CHUNKS = _ROWS_PER_W // _R
_NBUF = 3


@functools.partial(
    pl.kernel,
    out_type=jax.ShapeDtypeStruct((_N, _C), jnp.float32),
    mesh=plsc.VectorSubcoreMesh(core_axis_name="c", subcore_axis_name="s"),
    compiler_params=pltpu.CompilerParams(needs_layout_passes=False),
    scratch_types=(
        [pltpu.VMEM((_R, _C), jnp.float32) for _ in range(_NBUF)]
        + [
            pltpu.VMEM((_ROWS_PER_W,), jnp.int32),
            pltpu.VMEM((_ROWS_PER_W,), jnp.int32),
            pltpu.VMEM((_L,), jnp.int32),
        ]
        + [pltpu.SemaphoreType.DMA for _ in range(2 * _NBUF)]
    ),
)
def _sc_dispatch(tensor_2d, op_hbm, pred_hbm, wp_hbm, out_2d,
                 b0, b1, b2, opbuf, prbuf, wpv,
                 is0, is1, is2, os0, os1, os2):
    bufs = [b0, b1, b2]
    isems = [is0, is1, is2]
    osems = [os0, os1, os2]

    wid = lax.axis_index("s") * _NC + lax.axis_index("c")
    base = wid * _ROWS_PER_W

    pltpu.sync_copy(wp_hbm, wpv)
    pltpu.sync_copy(op_hbm.at[pl.ds(base, _ROWS_PER_W)], opbuf)
    pltpu.sync_copy(pred_hbm.at[pl.ds(base, _ROWS_PER_W)], prbuf)
    wp_vec = wpv[...]

    def start_in(g):
        s = g % _NBUF
        return pltpu.async_copy(
            tensor_2d.at[pl.ds(base + g * _R, _R)], bufs[s], isems[s])

    def start_out(g):
        s = g % _NBUF
        return pltpu.async_copy(
            bufs[s], out_2d.at[pl.ds(base + g * _R, _R)], osems[s])

    def compute(g):
        s = g % _NBUF
        buf = bufs[s]
        lbase = g * _R

        def vec_body(j, c2):
            opv = opbuf[pl.ds(lbase + j * _L, _L)]
            prv = prbuf[pl.ds(lbase + j * _L, _L)].astype(jnp.float32)
            pos = lax.gather(
                wp_vec, opv[:, None],
                lax.GatherDimensionNumbers(
                    offset_dims=(), collapsed_slice_dims=(0,),
                    start_index_map=(0,)),
                slice_sizes=(1,),
                mode=lax.GatherScatterMode.PROMISE_IN_BOUNDS)
            rows = lax.iota(jnp.int32, _L) + j * _L
            msk = opv < _NUM_WRITE_OPS
            plsc.store_scatter(buf, [rows, pos], prv, mask=msk)
            return c2

        lax.fori_loop(0, _R // _L, vec_body, 0)

    in_d = {}
    out_d = {}
    for g in range(min(_NBUF, _CHUNKS)):
        in_d[g] = start_in(g)
    for g in range(_CHUNKS):
        in_d[g].wait()
        compute(g)
        out_d[g] = start_out(g)
        if g + 2 < _CHUNKS and g >= 1:
            out_d[g - 1].wait()
            in_d[g + 2] = start_in(g + 2)
    for g in range(max(0, _CHUNKS - 2), _CHUNKS):
        out_d[g].wait()


def kernel(tensor, operation, prediction, write_positions):
    return _sc_dispatch(tensor, operation, prediction, write_positions)


# speedup vs baseline: 2.1305x; 1.0003x over previous
import functools

import jax
import jax.numpy as jnp
from jax import lax
from jax.experimental import pallas as pl
from jax.experimental.pallas import tpu as pltpu
from jax.experimental.pallas import tpu_sc as plsc

_N = 262144
_C = 64
_NUM_WRITE_OPS = 8

_NC = 2
_NS = 16
_NW = _NC * _NS
_L = 16

_ROWS_PER_W = _N // _NW
_R = 128
_CHUNKS = _ROWS_PER_W // _R
_NBUF = 6


@functools.partial(
    pl.kernel,
    out_type=jax.ShapeDtypeStruct((_N, _C), jnp.float32),
    mesh=plsc.VectorSubcoreMesh(core_axis_name="c", subcore_axis_name="s"),
    compiler_params=pltpu.CompilerParams(needs_layout_passes=False),
    scratch_types=(
        [pltpu.VMEM((_R, _C), jnp.float32) for _ in range(_NBUF)]
        + [
            pltpu.VMEM((_ROWS_PER_W,), jnp.int32),
            pltpu.VMEM((_ROWS_PER_W,), jnp.int32),
            pltpu.VMEM((_L,), jnp.int32),
        ]
        + [pltpu.SemaphoreType.DMA for _ in range(2 * _NBUF)]
    ),
)
def _sc_dispatch(tensor_2d, op_hbm, pred_hbm, wp_hbm, out_2d,
                 b0, b1, b2, b3, b4, b5, opbuf, prbuf, wpv,
                 is0, is1, is2, is3, is4, is5,
                 os0, os1, os2, os3, os4, os5):
    bufs = [b0, b1, b2, b3, b4, b5]
    isems = [is0, is1, is2, is3, is4, is5]
    osems = [os0, os1, os2, os3, os4, os5]

    wid = lax.axis_index("s") * _NC + lax.axis_index("c")
    base = wid * _ROWS_PER_W

    pltpu.sync_copy(wp_hbm, wpv)
    pltpu.sync_copy(op_hbm.at[pl.ds(base, _ROWS_PER_W)], opbuf)
    pltpu.sync_copy(pred_hbm.at[pl.ds(base, _ROWS_PER_W)], prbuf)
    wp_vec = wpv[...]

    def start_in(g):
        s = g % _NBUF
        return pltpu.async_copy(
            tensor_2d.at[pl.ds(base + g * _R, _R)], bufs[s], isems[s])

    def start_out(g):
        s = g % _NBUF
        return pltpu.async_copy(
            bufs[s], out_2d.at[pl.ds(base + g * _R, _R)], osems[s])

    def compute(g):
        s = g % _NBUF
        buf = bufs[s]
        lbase = g * _R

        def vec_body(j, c2):
            opv = opbuf[pl.ds(lbase + j * _L, _L)]
            prv = prbuf[pl.ds(lbase + j * _L, _L)].astype(jnp.float32)
            pos = lax.gather(
                wp_vec, opv[:, None],
                lax.GatherDimensionNumbers(
                    offset_dims=(), collapsed_slice_dims=(0,),
                    start_index_map=(0,)),
                slice_sizes=(1,),
                mode=lax.GatherScatterMode.PROMISE_IN_BOUNDS)
            rows = lax.iota(jnp.int32, _L) + j * _L
            msk = opv < _NUM_WRITE_OPS
            plsc.store_scatter(buf, [rows, pos], prv, mask=msk)
            return c2

        lax.fori_loop(0, _R // _L, vec_body, 0)

    in_d = {}
    out_d = {}
    for g in range(min(_NBUF, _CHUNKS)):
        in_d[g] = start_in(g)
    for g in range(_CHUNKS):
        in_d[g].wait()
        compute(g)
        out_d[g] = start_out(g)
        if g + _NBUF - 1 < _CHUNKS and g >= 1:
            out_d[g - 1].wait()
            in_d[g + _NBUF - 1] = start_in(g + _NBUF - 1)
    for g in range(max(0, _CHUNKS - _NBUF), _CHUNKS):
        out_d[g].wait()


def kernel(tensor, operation, prediction, write_positions):
    return _sc_dispatch(tensor, operation, prediction, write_positions)
